# baseline (speedup 1.0000x reference)
import functools

import numpy as np
import jax
import jax.numpy as jnp
from jax import lax
from jax.experimental import pallas as pl
from jax.experimental.pallas import tpu as pltpu
from jax.experimental.pallas import tpu_sc as plsc

K = 1024
CH = 256
B = 4
H = W = 32
N0 = H * W
N1 = N0 // 4
N2 = N0 // 16
R0 = B * N0

NC, NS, LANES = 2, 16, 16
NW = NC * NS
BPW = R0 // NW

HIGHEST = lax.Precision.HIGHEST


def _fold_rows(t):
    sz = t.shape[0]
    while sz > 1:
        sz //= 2
        t = t[:sz] + t[sz:]
    return t


def _match_body(x0_ref, x1_ref, x2s_ref, cb0_ref, cb1_ref, c20_ref, c21_ref,
                zsel_ref, zb_ref):
    c20 = c20_ref[...]
    c21 = c21_ref[...]

    def scale(xs, cb_ref, c2, need_p):
        n = xs.shape[1]
        x2row = _fold_rows(xs * xs)
        prod = lax.dot_general(cb_ref[...], xs, (((1,), (0,)), ((), ())),
                               preferred_element_type=jnp.float32)
        dist = x2row - 2.0 * prod + c2
        dmin = jnp.min(dist, axis=0, keepdims=True)
        ii = lax.broadcasted_iota(jnp.int32, (K, n), 0).astype(jnp.float32)
        idxrow = jnp.min(jnp.where(dist == dmin, ii, float(K)),
                         axis=0, keepdims=True)
        if not need_p:
            return None, idxrow
        denom = jnp.sum(jnp.exp(dmin - dist), axis=0, keepdims=True)
        return 1.0 / denom, idxrow

    p0, i0 = scale(x0_ref[0], cb0_ref, c20, True)
    p1, i1 = scale(x1_ref[0], cb0_ref, c20, True)
    p2, i2 = scale(x2s_ref[0], cb0_ref, c20, True)
    _, zb = scale(x0_ref[0], cb1_ref, c21, False)

    def up_onehot(n_coarse, n_fine, wsrc, wdst):
        q = lax.broadcasted_iota(jnp.int32, (1, n_fine), 1)
        m = (q // wdst // 2) * wsrc + (q % wdst) // 2
        r = lax.broadcasted_iota(jnp.int32, (n_coarse, n_fine), 0)
        return (r == m).astype(jnp.float32)

    u2 = up_onehot(N2, N1, W // 4, W // 2)
    s2u = lax.dot_general(jnp.concatenate([p2, i2], axis=0),
                          u2, (((1,), (0,)), ((), ())),
                          precision=HIGHEST, preferred_element_type=jnp.float32)
    c12 = s2u[0:1] > p1
    pw = jnp.where(c12, s2u[0:1], p1)
    iwf = jnp.where(c12, s2u[1:2], i1)
    u1 = up_onehot(N1, N0, W // 2, W)
    s1u = lax.dot_general(jnp.concatenate([pw, iwf], axis=0), u1,
                          (((1,), (0,)), ((), ())),
                          precision=HIGHEST, preferred_element_type=jnp.float32)
    c01 = s1u[0:1] > p0
    zself = jnp.where(c01, s1u[1:2], i0)
    zsel_ref[...] = jnp.reshape((zself + 0.5).astype(jnp.int32), (N0,))
    zb_ref[...] = jnp.reshape(zb.astype(jnp.int32), (N0,))


def _tc_match(x0, x1, x2s, cb0, cb1, c20, c21):
    full = lambda shape: pl.BlockSpec(shape, lambda b: (0,) * len(shape))
    per_b = lambda shape: pl.BlockSpec((1,) + shape, lambda b: (b, 0, 0))
    lin = lambda n: pl.BlockSpec((n,), lambda b: (b,))
    return pl.pallas_call(
        _match_body,
        grid=(B,),
        in_specs=[
            per_b((CH, N0)), per_b((CH, N1)), per_b((CH, N2)),
            full((K, CH)), full((K, CH)), full((K, 1)), full((K, 1)),
        ],
        out_specs=[lin(N0), lin(N0)],
        out_shape=[
            jax.ShapeDtypeStruct((R0,), jnp.int32),
            jax.ShapeDtypeStruct((R0,), jnp.int32),
        ],
    )(x0, x1, x2s, cb0, cb1, c20, c21)


def _sc_body(zs_h, zb_h, cb0_h, cb1_h, q_h,
             zs_v, zb_v, rows0_v, rows1_v, sem_a, sem_b):
    wid = lax.axis_index("s") * NC + lax.axis_index("c")
    base = wid * BPW
    pltpu.sync_copy(zs_h.at[pl.ds(base, BPW)], zs_v)
    cp_a = pltpu.async_copy(cb0_h.at[zs_v], rows0_v, sem_a)
    pltpu.sync_copy(zb_h.at[pl.ds(base, BPW)], zb_v)
    cp_b = pltpu.async_copy(cb1_h.at[zb_v], rows1_v, sem_b)
    cp_a.wait()
    cp_b.wait()

    @plsc.parallel_loop(0, BPW, 1, unroll=8)
    def _add_row(r):
        for c in range(CH // LANES):
            s2 = pl.ds(c * LANES, LANES)
            rows0_v[r, s2] = (rows0_v[r, s2] + rows1_v[r, s2]) * 0.5

    pltpu.sync_copy(rows0_v, q_h.at[pl.ds(base, BPW)])


def _sc_combine(zsel, zb, cb0, cb1):
    mesh = plsc.VectorSubcoreMesh(core_axis_name="c", subcore_axis_name="s",
                                  num_cores=NC, num_subcores=NS)
    fn = pl.kernel(
        _sc_body,
        out_type=jax.ShapeDtypeStruct((R0, CH), jnp.float32),
        mesh=mesh,
        scratch_types=[
            pltpu.VMEM((BPW,), jnp.int32),
            pltpu.VMEM((BPW,), jnp.int32),
            pltpu.VMEM((BPW, CH), jnp.float32),
            pltpu.VMEM((BPW, CH), jnp.float32),
            pltpu.SemaphoreType.DMA,
            pltpu.SemaphoreType.DMA,
        ],
    )
    return fn(zsel, zb, cb0, cb1)


def kernel(input, cb0, cb1, cb2, cb3):
    b, c, h, w = input.shape
    r1 = jax.image.resize(input, (b, c, h // 2, w // 2), method='bilinear')
    r2 = jax.image.resize(input, (b, c, h // 4, w // 4), method='bilinear')
    x0 = input.reshape(B, CH, N0)
    x1 = r1.reshape(B, CH, N1)
    x2s = r2.reshape(B, CH, N2)

    c20 = jnp.sum(cb0 * cb0, axis=-1).reshape(K, 1)
    c21 = jnp.sum(cb1 * cb1, axis=-1).reshape(K, 1)
    zsel, zb = _tc_match(x0, x1, x2s, cb0, cb1, c20, c21)
    q = _sc_combine(zsel, zb, cb0, cb1)

    zidx0 = jnp.stack([zsel.reshape(b, h, w), zb.reshape(b, h, w)], axis=1)
    quant0 = jnp.transpose(q.reshape(b, h, w, CH), (0, 3, 1, 2))
    return input, zidx0, quant0


# speedup vs baseline: 1.0179x; 1.0179x over previous
import functools

import numpy as np
import jax
import jax.numpy as jnp
from jax import lax
from jax.experimental import pallas as pl
from jax.experimental.pallas import tpu as pltpu
from jax.experimental.pallas import tpu_sc as plsc

K = 1024
CH = 256
B = 4
H = W = 32
N0 = H * W
N1 = N0 // 4
N2 = N0 // 16
R0 = B * N0

NC, NS, LANES = 2, 16, 16
NW = NC * NS
BPW = R0 // NW

HIGHEST = lax.Precision.HIGHEST
BPG = 2


def _fold_rows(t):
    sz = t.shape[0]
    while sz > 1:
        sz //= 2
        t = t[:sz] + t[sz:]
    return t


def _match_body(x0_ref, x1_ref, x2s_ref, cb0_ref, cb1_ref, c20_ref, c21_ref,
                zsel_ref, zb_ref):
    c20 = c20_ref[...]
    c21 = c21_ref[...]

    def scale(xs, cb_ref, c2, need_p):
        n = xs.shape[1]
        x2row = _fold_rows(xs * xs)
        prod = lax.dot_general(cb_ref[...], xs, (((1,), (0,)), ((), ())),
                               preferred_element_type=jnp.float32)
        dist = x2row - 2.0 * prod + c2
        dmin = jnp.min(dist, axis=0, keepdims=True)
        ii = lax.broadcasted_iota(jnp.int32, (K, n), 0).astype(jnp.float32)
        idxrow = jnp.min(jnp.where(dist == dmin, ii, float(K)),
                         axis=0, keepdims=True)
        if not need_p:
            return None, idxrow
        denom = jnp.sum(jnp.exp(dmin - dist), axis=0, keepdims=True)
        return 1.0 / denom, idxrow

    for g in range(BPG):
        _per_image(g, x0_ref, x1_ref, x2s_ref, cb0_ref, cb1_ref, c20, c21,
                   scale, zsel_ref, zb_ref)


def _per_image(g, x0_ref, x1_ref, x2s_ref, cb0_ref, cb1_ref, c20, c21,
               scale, zsel_ref, zb_ref):
    p0, i0 = scale(x0_ref[g], cb0_ref, c20, True)
    p1, i1 = scale(x1_ref[g], cb0_ref, c20, True)
    p2, i2 = scale(x2s_ref[g], cb0_ref, c20, True)
    _, zb = scale(x0_ref[g], cb1_ref, c21, False)

    def up_onehot(n_coarse, n_fine, wsrc, wdst):
        q = lax.broadcasted_iota(jnp.int32, (1, n_fine), 1)
        m = (q // wdst // 2) * wsrc + (q % wdst) // 2
        r = lax.broadcasted_iota(jnp.int32, (n_coarse, n_fine), 0)
        return (r == m).astype(jnp.float32)

    u2 = up_onehot(N2, N1, W // 4, W // 2)
    s2u = lax.dot_general(jnp.concatenate([p2, i2], axis=0),
                          u2, (((1,), (0,)), ((), ())),
                          precision=HIGHEST, preferred_element_type=jnp.float32)
    c12 = s2u[0:1] > p1
    pw = jnp.where(c12, s2u[0:1], p1)
    iwf = jnp.where(c12, s2u[1:2], i1)
    u1 = up_onehot(N1, N0, W // 2, W)
    s1u = lax.dot_general(jnp.concatenate([pw, iwf], axis=0), u1,
                          (((1,), (0,)), ((), ())),
                          precision=HIGHEST, preferred_element_type=jnp.float32)
    c01 = s1u[0:1] > p0
    zself = jnp.where(c01, s1u[1:2], i0)
    zsel_ref[pl.ds(g * N0, N0)] = jnp.reshape(
        (zself + 0.5).astype(jnp.int32), (N0,))
    zb_ref[pl.ds(g * N0, N0)] = jnp.reshape(zb.astype(jnp.int32), (N0,))


def _tc_match(x0, x1, x2s, cb0, cb1, c20, c21):
    full = lambda shape: pl.BlockSpec(shape, lambda b: (0,) * len(shape))
    per_b = lambda shape: pl.BlockSpec(shape, lambda b: (b, 0, 0))
    lin = lambda n: pl.BlockSpec((n,), lambda b: (b,))
    return pl.pallas_call(
        _match_body,
        grid=(B // BPG,),
        in_specs=[
            per_b((BPG, CH, N0)), per_b((BPG, CH, N1)), per_b((BPG, CH, N2)),
            full((K, CH)), full((K, CH)), full((K, 1)), full((K, 1)),
        ],
        out_specs=[lin(BPG * N0), lin(BPG * N0)],
        out_shape=[
            jax.ShapeDtypeStruct((R0,), jnp.int32),
            jax.ShapeDtypeStruct((R0,), jnp.int32),
        ],
    )(x0, x1, x2s, cb0, cb1, c20, c21)


def _sc_body(zs_h, zb_h, cb0_h, cb1_h, q_h,
             zs_v, zb_v, rows0_v, rows1_v, sem_a, sem_b):
    wid = lax.axis_index("s") * NC + lax.axis_index("c")
    base = wid * BPW
    pltpu.sync_copy(zs_h.at[pl.ds(base, BPW)], zs_v)
    cp_a = pltpu.async_copy(cb0_h.at[zs_v], rows0_v, sem_a)
    pltpu.sync_copy(zb_h.at[pl.ds(base, BPW)], zb_v)
    cp_b = pltpu.async_copy(cb1_h.at[zb_v], rows1_v, sem_b)
    cp_a.wait()
    cp_b.wait()

    @plsc.parallel_loop(0, BPW, 1, unroll=8)
    def _add_row(r):
        for c in range(CH // LANES):
            s2 = pl.ds(c * LANES, LANES)
            rows0_v[r, s2] = (rows0_v[r, s2] + rows1_v[r, s2]) * 0.5

    pltpu.sync_copy(rows0_v, q_h.at[pl.ds(base, BPW)])


def _sc_combine(zsel, zb, cb0, cb1):
    mesh = plsc.VectorSubcoreMesh(core_axis_name="c", subcore_axis_name="s",
                                  num_cores=NC, num_subcores=NS)
    fn = pl.kernel(
        _sc_body,
        out_type=jax.ShapeDtypeStruct((R0, CH), jnp.float32),
        mesh=mesh,
        scratch_types=[
            pltpu.VMEM((BPW,), jnp.int32),
            pltpu.VMEM((BPW,), jnp.int32),
            pltpu.VMEM((BPW, CH), jnp.float32),
            pltpu.VMEM((BPW, CH), jnp.float32),
            pltpu.SemaphoreType.DMA,
            pltpu.SemaphoreType.DMA,
        ],
    )
    return fn(zsel, zb, cb0, cb1)


def kernel(input, cb0, cb1, cb2, cb3):
    b, c, h, w = input.shape
    r1 = jax.image.resize(input, (b, c, h // 2, w // 2), method='bilinear')
    r2 = jax.image.resize(input, (b, c, h // 4, w // 4), method='bilinear')
    x0 = input.reshape(B, CH, N0)
    x1 = r1.reshape(B, CH, N1)
    x2s = r2.reshape(B, CH, N2)

    c20 = jnp.sum(cb0 * cb0, axis=-1).reshape(K, 1)
    c21 = jnp.sum(cb1 * cb1, axis=-1).reshape(K, 1)
    zsel, zb = _tc_match(x0, x1, x2s, cb0, cb1, c20, c21)
    q = _sc_combine(zsel, zb, cb0, cb1)

    zidx0 = jnp.stack([zsel.reshape(b, h, w), zb.reshape(b, h, w)], axis=1)
    quant0 = jnp.transpose(q.reshape(b, h, w, CH), (0, 3, 1, 2))
    return input, zidx0, quant0


# speedup vs baseline: 1.0192x; 1.0012x over previous
import jax
import jax.numpy as jnp
from jax import lax
from jax.experimental import pallas as pl
from jax.experimental.pallas import tpu as pltpu
from jax.experimental.pallas import tpu_sc as plsc

K = 1024
CH = 256
B = 4
H = W = 32
N0 = H * W
N1 = N0 // 4
N2 = N0 // 16
R0 = B * N0

NC, NS, LANES = 2, 16, 16
NW = NC * NS
BPW = R0 // NW

HIGHEST = lax.Precision.HIGHEST
BPG = 2


def _fold_rows(t):
    sz = t.shape[0]
    while sz > 1:
        sz //= 2
        t = t[:sz] + t[sz:]
    return t


def _match_body(x0_ref, x1_ref, x2s_ref, cb0_ref, cb1_ref, c20_ref, c21_ref,
                zsel_ref, zb_ref):
    c20 = c20_ref[...]
    c21 = c21_ref[...]

    def scale(xs, cb_ref, c2, need_p):
        n = xs.shape[1]
        x2row = _fold_rows(xs * xs)
        prod = lax.dot_general(cb_ref[...], xs, (((1,), (0,)), ((), ())),
                               preferred_element_type=jnp.float32)
        dist = x2row - 2.0 * prod + c2
        dmin = jnp.min(dist, axis=0, keepdims=True)
        ii = lax.broadcasted_iota(jnp.int32, (K, n), 0).astype(jnp.float32)
        idxrow = jnp.min(jnp.where(dist == dmin, ii, float(K)),
                         axis=0, keepdims=True)
        if not need_p:
            return None, idxrow
        denom = jnp.sum(jnp.exp(dmin - dist), axis=0, keepdims=True)
        return 1.0 / denom, idxrow

    for g in range(BPG):
        _per_image(g, x0_ref, x1_ref, x2s_ref, cb0_ref, cb1_ref, c20, c21,
                   scale, zsel_ref, zb_ref)


def _per_image(g, x0_ref, x1_ref, x2s_ref, cb0_ref, cb1_ref, c20, c21,
               scale, zsel_ref, zb_ref):
    p0, i0 = scale(x0_ref[g], cb0_ref, c20, True)
    p1, i1 = scale(x1_ref[g], cb0_ref, c20, True)
    p2, i2 = scale(x2s_ref[g], cb0_ref, c20, True)
    _, zb = scale(x0_ref[g], cb1_ref, c21, False)

    def up_onehot(n_coarse, n_fine, wsrc, wdst):
        q = lax.broadcasted_iota(jnp.int32, (1, n_fine), 1)
        m = (q // wdst // 2) * wsrc + (q % wdst) // 2
        r = lax.broadcasted_iota(jnp.int32, (n_coarse, n_fine), 0)
        return (r == m).astype(jnp.float32)

    u2 = up_onehot(N2, N1, W // 4, W // 2)
    s2u = lax.dot_general(jnp.concatenate([p2, i2], axis=0),
                          u2, (((1,), (0,)), ((), ())),
                          precision=HIGHEST, preferred_element_type=jnp.float32)
    c12 = s2u[0:1] > p1
    pw = jnp.where(c12, s2u[0:1], p1)
    iwf = jnp.where(c12, s2u[1:2], i1)
    u1 = up_onehot(N1, N0, W // 2, W)
    s1u = lax.dot_general(jnp.concatenate([pw, iwf], axis=0), u1,
                          (((1,), (0,)), ((), ())),
                          precision=HIGHEST, preferred_element_type=jnp.float32)
    c01 = s1u[0:1] > p0
    zself = jnp.where(c01, s1u[1:2], i0)
    zsel_ref[pl.ds(g * N0, N0)] = jnp.reshape(
        (zself + 0.5).astype(jnp.int32), (N0,))
    zb_ref[pl.ds(g * N0, N0)] = jnp.reshape(zb.astype(jnp.int32), (N0,))


def _tc_match(x0, x1, x2s, cb0, cb1, c20, c21):
    full = lambda shape: pl.BlockSpec(shape, lambda b: (0,) * len(shape))
    per_b = lambda shape: pl.BlockSpec(shape, lambda b: (b, 0, 0))
    lin = lambda n: pl.BlockSpec((n,), lambda b: (b,))
    return pl.pallas_call(
        _match_body,
        grid=(B // BPG,),
        in_specs=[
            per_b((BPG, CH, N0)), per_b((BPG, CH, N1)), per_b((BPG, CH, N2)),
            full((K, CH)), full((K, CH)), full((K, 1)), full((K, 1)),
        ],
        out_specs=[lin(BPG * N0), lin(BPG * N0)],
        out_shape=[
            jax.ShapeDtypeStruct((R0,), jnp.int32),
            jax.ShapeDtypeStruct((R0,), jnp.int32),
        ],
    )(x0, x1, x2s, cb0, cb1, c20, c21)


def _sc_body(zs_h, zb_h, cb0_h, cb1_h, q_h,
             zs_v, zb_v, rows0_v, rows1_v, sem_a, sem_b):
    wid = lax.axis_index("s") * NC + lax.axis_index("c")
    base = wid * BPW
    pltpu.sync_copy(zs_h.at[pl.ds(base, BPW)], zs_v)
    cp_a = pltpu.async_copy(cb0_h.at[zs_v], rows0_v, sem_a)
    pltpu.sync_copy(zb_h.at[pl.ds(base, BPW)], zb_v)
    cp_b = pltpu.async_copy(cb1_h.at[zb_v], rows1_v, sem_b)
    cp_a.wait()
    cp_b.wait()

    @plsc.parallel_loop(0, BPW, 1, unroll=8)
    def _add_row(r):
        for c in range(CH // LANES):
            s2 = pl.ds(c * LANES, LANES)
            rows0_v[r, s2] = (rows0_v[r, s2] + rows1_v[r, s2]) * 0.5

    pltpu.sync_copy(rows0_v, q_h.at[pl.ds(base, BPW)])


def _sc_combine(zsel, zb, cb0, cb1):
    mesh = plsc.VectorSubcoreMesh(core_axis_name="c", subcore_axis_name="s",
                                  num_cores=NC, num_subcores=NS)
    fn = pl.kernel(
        _sc_body,
        out_type=jax.ShapeDtypeStruct((R0, CH), jnp.float32),
        mesh=mesh,
        scratch_types=[
            pltpu.VMEM((BPW,), jnp.int32),
            pltpu.VMEM((BPW,), jnp.int32),
            pltpu.VMEM((BPW, CH), jnp.float32),
            pltpu.VMEM((BPW, CH), jnp.float32),
            pltpu.SemaphoreType.DMA,
            pltpu.SemaphoreType.DMA,
        ],
    )
    return fn(zsel, zb, cb0, cb1)


def kernel(input, cb0, cb1, cb2, cb3):
    b, c, h, w = input.shape
    r1 = jax.image.resize(input, (b, c, h // 2, w // 2), method='bilinear')
    r2 = jax.image.resize(input, (b, c, h // 4, w // 4), method='bilinear')
    x0 = input.reshape(B, CH, N0)
    x1 = r1.reshape(B, CH, N1)
    x2s = r2.reshape(B, CH, N2)

    c20 = jnp.sum(cb0 * cb0, axis=-1).reshape(K, 1)
    c21 = jnp.sum(cb1 * cb1, axis=-1).reshape(K, 1)
    zsel, zb = _tc_match(x0, x1, x2s, cb0, cb1, c20, c21)
    q = _sc_combine(zsel, zb, cb0, cb1)

    zidx0 = jnp.stack([zsel.reshape(b, h, w), zb.reshape(b, h, w)], axis=1)
    quant0 = jnp.transpose(q.reshape(b, h, w, CH), (0, 3, 1, 2))
    return input, zidx0, quant0
